# Initial kernel scaffold; baseline (speedup 1.0000x reference)
import jax
import jax.numpy as jnp
from jax.experimental import pallas as pl


def kernel(J, b, W_m1, b_m1, W_m2, b_m2, W_m3, b_m3, W_ih, b_ih, W_hh, b_hh, W_r1, b_r1, W_r2, b_r2, W_r3, b_r3):
    raise NotImplementedError("write your pallas kernel here")


# speedup vs baseline: 117.1385x; 117.1385x over previous
import functools

import jax
import jax.numpy as jnp
from jax import lax
from jax.experimental import pallas as pl
from jax.experimental.pallas import tpu as pltpu
from jax.experimental.pallas import tpu_sc as plsc

N = 2048
SD = 64
HM = 128
N_STEPS = 10
CAP = 36864
NC = 2
NS = 16
NT = NC * NS
EPT = CAP // NT
CHUNK = 128
NCH = EPT // CHUNK
N_ACC = N + 128
ZPT = N_ACC // NS
OPT = N // NS

_f32 = jnp.float32


def _sc_gather_body(a_hbm, b_hbm, ridx_hbm, cidx_hbm, ar_out, bc_out,
                    ridx_v, cidx_v, bufr, bufc, semr, semc):
    wid = lax.axis_index("s") * NC + lax.axis_index("c")
    base = wid * EPT

    def chunk(j, carry):
        pltpu.sync_copy(ridx_hbm.at[wid, j], ridx_v)
        pltpu.sync_copy(cidx_hbm.at[wid, j], cidx_v)
        cr = pltpu.async_copy(a_hbm.at[ridx_v], bufr, semr)
        cc = pltpu.async_copy(b_hbm.at[cidx_v], bufc, semc)
        cr.wait()
        cc.wait()
        pltpu.sync_copy(bufr, ar_out.at[pl.ds(base + j * CHUNK, CHUNK)])
        pltpu.sync_copy(bufc, bc_out.at[pl.ds(base + j * CHUNK, CHUNK)])
        return carry

    lax.fori_loop(0, NCH, chunk, 0)


def _sc_scatter_body(msgs_hbm, sidx_hbm, zeros_hbm, out_hbm,
                     sidx_v, bufm, acc_sh, sem):
    cid = lax.axis_index("c")
    sid = lax.axis_index("s")
    wid = sid * NC + cid
    base = wid * EPT
    pltpu.sync_copy(zeros_hbm.at[pl.ds(sid * ZPT, ZPT)],
                    acc_sh.at[pl.ds(sid * ZPT, ZPT)])
    plsc.subcore_barrier()

    def chunk(j, carry):
        pltpu.sync_copy(sidx_hbm.at[wid, j], sidx_v)
        pltpu.sync_copy(msgs_hbm.at[pl.ds(base + j * CHUNK, CHUNK)], bufm)
        pltpu.sync_copy(bufm, acc_sh.at[sidx_v], add=True)
        return carry

    lax.fori_loop(0, NCH, chunk, 0)
    plsc.subcore_barrier()
    pltpu.sync_copy(acc_sh.at[pl.ds(sid * OPT, OPT)],
                    out_hbm.at[cid, pl.ds(sid * OPT, OPT)])


def _make_sc_calls():
    mesh = plsc.VectorSubcoreMesh(core_axis_name="c", subcore_axis_name="s",
                                  num_cores=NC, num_subcores=NS)
    gather = pl.kernel(
        _sc_gather_body,
        out_type=(jax.ShapeDtypeStruct((CAP, HM), _f32),
                  jax.ShapeDtypeStruct((CAP, HM), _f32)),
        mesh=mesh,
        scratch_types=[
            pltpu.VMEM((CHUNK,), jnp.int32),
            pltpu.VMEM((CHUNK,), jnp.int32),
            pltpu.VMEM((CHUNK, HM), _f32),
            pltpu.VMEM((CHUNK, HM), _f32),
            pltpu.SemaphoreType.DMA,
            pltpu.SemaphoreType.DMA,
        ],
    )
    scatter = pl.kernel(
        _sc_scatter_body,
        out_type=jax.ShapeDtypeStruct((NC, N, HM), _f32),
        mesh=mesh,
        scratch_types=[
            pltpu.VMEM((CHUNK,), jnp.int32),
            pltpu.VMEM((CHUNK, HM), _f32),
            pltpu.VMEM_SHARED((N_ACC, HM), _f32),
            pltpu.SemaphoreType.DMA,
        ],
    )
    return gather, scatter


EB = 2304


def _mlp_body(ar, br_, ef, w1c, b1, w2, b2, w3, b3, out):
    x = ar[...] + br_[...] + jnp.dot(ef[...], w1c[...], preferred_element_type=_f32)
    x = jnp.maximum(x + b1[...], 0.0)
    x = jnp.maximum(jnp.dot(x, w2[...], preferred_element_type=_f32) + b2[...], 0.0)
    out[...] = jnp.dot(x, w3[...], preferred_element_type=_f32) + b3[...]


def _gru_body(nm2, h, wir, wiz, win, whr, whz, whn, br, bz, bn, cr, cz, cn,
              w1a, w1b, out, a_out, b_out):
    x = (nm2[0] + nm2[1])[:, :SD]
    hh = h[...]
    r = jax.nn.sigmoid(jnp.dot(x, wir[...], preferred_element_type=_f32)
                       + jnp.dot(hh, whr[...], preferred_element_type=_f32)
                       + br[...] + cr[...])
    z = jax.nn.sigmoid(jnp.dot(x, wiz[...], preferred_element_type=_f32)
                       + jnp.dot(hh, whz[...], preferred_element_type=_f32)
                       + bz[...] + cz[...])
    n = jnp.tanh(jnp.dot(x, win[...], preferred_element_type=_f32)
                 + bn[...]
                 + r * (jnp.dot(hh, whn[...], preferred_element_type=_f32) + cn[...]))
    hnew = (1.0 - z) * n + z * hh
    out[...] = hnew
    a_out[...] = jnp.dot(hnew, w1a[...], preferred_element_type=_f32)
    b_out[...] = jnp.dot(hnew, w1b[...], preferred_element_type=_f32)


def _readout_body(h, w1, b1, w2, b2, wd, bd, out):
    x = jnp.maximum(jnp.dot(h[...], w1[...], preferred_element_type=_f32) + b1[...], 0.0)
    x = jnp.maximum(jnp.dot(x, w2[...], preferred_element_type=_f32) + b2[...], 0.0)
    d = jnp.sum(x * wd[...], axis=1, keepdims=True) + bd[...]
    sgn = 1.0 - 2.0 * lax.broadcasted_iota(jnp.int32, (N, 2), 1).astype(_f32)
    out[...] = jax.nn.sigmoid(sgn * d)


def _make_tc_calls():
    full = pl.BlockSpec(index_map=lambda i: (0, 0))
    mlp = pl.pallas_call(
        _mlp_body,
        grid=(CAP // EB,),
        in_specs=[
            pl.BlockSpec((EB, HM), lambda i: (i, 0)),
            pl.BlockSpec((EB, HM), lambda i: (i, 0)),
            pl.BlockSpec((EB, 4), lambda i: (i, 0)),
            full, full, full, full, full, full,
        ],
        out_specs=pl.BlockSpec((EB, HM), lambda i: (i, 0)),
        out_shape=jax.ShapeDtypeStruct((CAP, HM), _f32),
    )
    gru = pl.pallas_call(
        _gru_body,
        out_shape=(jax.ShapeDtypeStruct((N, SD), _f32),
                   jax.ShapeDtypeStruct((N, HM), _f32),
                   jax.ShapeDtypeStruct((N, HM), _f32)),
    )
    readout = pl.pallas_call(
        _readout_body,
        out_shape=jax.ShapeDtypeStruct((N, 2), _f32),
    )
    return mlp, gru, readout


def kernel(J, b, W_m1, b_m1, W_m2, b_m2, W_m3, b_m3, W_ih, b_ih, W_hh, b_hh,
           W_r1, b_r1, W_r2, b_r2, W_r3, b_r3):
    flat = J.reshape(-1)
    (eidx,) = jnp.nonzero(flat, size=CAP, fill_value=0)
    cnt = jnp.count_nonzero(flat)
    valid = jnp.arange(CAP) < cnt
    row = (eidx // N).astype(jnp.int32)
    col = (eidx - row * N).astype(jnp.int32)
    vf = valid.astype(_f32)[:, None]
    ef = jnp.stack([b[row], b[col], flat[eidx], J[col, row]], axis=-1) * vf
    sidx = jnp.where(valid, col, N).astype(jnp.int32)

    ridx3 = row.reshape(NT, NCH, CHUNK)
    cidx3 = col.reshape(NT, NCH, CHUNK)
    sidx3 = sidx.reshape(NT, NCH, CHUNK)
    zeros_acc = jnp.zeros((N_ACC, HM), _f32)

    w1a = W_m1[:, 0:SD].T
    w1b = W_m1[:, SD:2 * SD].T
    w1c = W_m1[:, 2 * SD:2 * SD + 4].T
    b1 = b_m1.reshape(1, HM)
    w2 = W_m2.T
    b2 = b_m2.reshape(1, HM)
    w3 = jnp.pad(W_m3.T, ((0, 0), (0, HM - SD)))
    b3 = jnp.pad(b_m3.reshape(1, SD), ((0, 0), (0, HM - SD)))
    wir, wiz, win = (W_ih[0:SD].T, W_ih[SD:2 * SD].T, W_ih[2 * SD:].T)
    whr, whz, whn = (W_hh[0:SD].T, W_hh[SD:2 * SD].T, W_hh[2 * SD:].T)
    br, bz, bn = (b_ih[0:SD].reshape(1, SD), b_ih[SD:2 * SD].reshape(1, SD),
                  b_ih[2 * SD:].reshape(1, SD))
    cr, cz, cn = (b_hh[0:SD].reshape(1, SD), b_hh[SD:2 * SD].reshape(1, SD),
                  b_hh[2 * SD:].reshape(1, SD))
    wr1 = W_r1.T
    br1 = b_r1.reshape(1, -1)
    wr2 = W_r2.T
    br2 = b_r2.reshape(1, -1)
    wd = (W_r3[0] - W_r3[1]).reshape(1, -1)
    bd = (b_r3[0] - b_r3[1]).reshape(1, 1)

    sc_gather, sc_scatter = _make_sc_calls()
    mlp, gru, readout = _make_tc_calls()

    def step(carry, _):
        hidden, A, B = carry
        ar, bc = sc_gather(A, B, ridx3, cidx3)
        msgs = mlp(ar, bc, ef, w1c, b1, w2, b2, w3, b3)
        nm2 = sc_scatter(msgs, sidx3, zeros_acc)
        hidden, A, B = gru(nm2, hidden, wir, wiz, win, whr, whz, whn,
                           br, bz, bn, cr, cz, cn, w1a, w1b)
        return (hidden, A, B), None

    hidden = jnp.zeros((N, SD), _f32)
    A0 = jnp.zeros((N, HM), _f32)
    B0 = jnp.zeros((N, HM), _f32)
    (hidden, _, _), _ = lax.scan(step, (hidden, A0, B0), None, length=N_STEPS)
    return readout(hidden, wr1, br1, wr2, br2, wd, bd)


# speedup vs baseline: 166.2373x; 1.4192x over previous
import functools

import jax
import jax.numpy as jnp
from jax import lax
from jax.experimental import pallas as pl
from jax.experimental.pallas import tpu as pltpu
from jax.experimental.pallas import tpu_sc as plsc

N = 2048
SD = 64
HM = 128
N_STEPS = 10
CAP = 40960
NC = 2
NS = 16
NT = NC * NS
EPT = CAP // NT
CHUNK = 128
NCH = EPT // CHUNK
N_ACC = N + 128
ZPT = N_ACC // NS
OPT = N // NS

_f32 = jnp.float32


def _sc_gather_body(a_hbm, b_hbm, ridx_hbm, cidx_hbm, ar_out, bc_out,
                    ridx_v, cidx_v, bufr0, bufr1, bufc0, bufc1, sem0, sem1):
    wid = lax.axis_index("s") * NC + lax.axis_index("c")
    base = wid * EPT
    pltpu.sync_copy(ridx_hbm.at[wid], ridx_v)
    pltpu.sync_copy(cidx_hbm.at[wid], cidx_v)
    bufr = (bufr0, bufr1)
    bufc = (bufc0, bufc1)
    sem = (sem0, sem1)
    pend = [None, None]

    def drain(j):
        cr, cc = pend[j & 1]
        cr.wait()
        cc.wait()
        off = base + j * CHUNK
        pltpu.sync_copy(bufr[j & 1], ar_out.at[pl.ds(off, CHUNK)])
        pltpu.sync_copy(bufc[j & 1], bc_out.at[pl.ds(off, CHUNK)])

    for j in range(NCH):
        p = j & 1
        cr = pltpu.async_copy(a_hbm.at[ridx_v.at[j]], bufr[p], sem[p])
        cc = pltpu.async_copy(b_hbm.at[cidx_v.at[j]], bufc[p], sem[p])
        pend[p] = (cr, cc)
        if j > 0:
            drain(j - 1)
    drain(NCH - 1)


def _sc_scatter_body(msgs_hbm, sidx_hbm, zeros_hbm, out_hbm,
                     sidx_v, bufm0, bufm1, acc_sh, sem0, sem1):
    cid = lax.axis_index("c")
    sid = lax.axis_index("s")
    wid = sid * NC + cid
    base = wid * EPT
    pltpu.sync_copy(zeros_hbm.at[pl.ds(sid * ZPT, ZPT)],
                    acc_sh.at[pl.ds(sid * ZPT, ZPT)])
    plsc.subcore_barrier()

    pltpu.sync_copy(sidx_hbm.at[wid], sidx_v)
    bufm = (bufm0, bufm1)
    sem = (sem0, sem1)
    pend = [None, None]
    pend[0] = pltpu.async_copy(msgs_hbm.at[pl.ds(base, CHUNK)], bufm[0], sem0)
    for j in range(NCH):
        p = j & 1
        if j < NCH - 1:
            off = base + (j + 1) * CHUNK
            pend[1 - p] = pltpu.async_copy(
                msgs_hbm.at[pl.ds(off, CHUNK)], bufm[1 - p], sem[1 - p])
        pend[p].wait()
        pltpu.sync_copy(bufm[p], acc_sh.at[sidx_v.at[j]], add=True)
    plsc.subcore_barrier()
    pltpu.sync_copy(acc_sh.at[pl.ds(sid * OPT, OPT)],
                    out_hbm.at[cid, pl.ds(sid * OPT, OPT)])


def _make_sc_calls():
    mesh = plsc.VectorSubcoreMesh(core_axis_name="c", subcore_axis_name="s",
                                  num_cores=NC, num_subcores=NS)
    gather = pl.kernel(
        _sc_gather_body,
        out_type=(jax.ShapeDtypeStruct((CAP, HM), _f32),
                  jax.ShapeDtypeStruct((CAP, HM), _f32)),
        mesh=mesh,
        scratch_types=[
            pltpu.VMEM((NCH, CHUNK), jnp.int32),
            pltpu.VMEM((NCH, CHUNK), jnp.int32),
            pltpu.VMEM((CHUNK, HM), _f32),
            pltpu.VMEM((CHUNK, HM), _f32),
            pltpu.VMEM((CHUNK, HM), _f32),
            pltpu.VMEM((CHUNK, HM), _f32),
            pltpu.SemaphoreType.DMA,
            pltpu.SemaphoreType.DMA,
        ],
    )
    scatter = pl.kernel(
        _sc_scatter_body,
        out_type=jax.ShapeDtypeStruct((NC, N, HM), _f32),
        mesh=mesh,
        scratch_types=[
            pltpu.VMEM((NCH, CHUNK), jnp.int32),
            pltpu.VMEM((CHUNK, HM), _f32),
            pltpu.VMEM((CHUNK, HM), _f32),
            pltpu.VMEM_SHARED((N_ACC, HM), _f32),
            pltpu.SemaphoreType.DMA,
            pltpu.SemaphoreType.DMA,
        ],
    )
    return gather, scatter


EB = 2560


def _mlp_body(ar, br_, ef, w1c, b1, w2, b2, w3, b3, out):
    x = ar[...] + br_[...] + jnp.dot(ef[...], w1c[...], preferred_element_type=_f32)
    x = jnp.maximum(x + b1[...], 0.0)
    x = jnp.maximum(jnp.dot(x, w2[...], preferred_element_type=_f32) + b2[...], 0.0)
    out[...] = jnp.dot(x, w3[...], preferred_element_type=_f32) + b3[...]


def _gru_body(nm2, h, wir, wiz, win, whr, whz, whn, br, bz, bn, cr, cz, cn,
              w1a, w1b, out, a_out, b_out):
    x = (nm2[0] + nm2[1])[:, :SD]
    hh = h[...]
    r = jax.nn.sigmoid(jnp.dot(x, wir[...], preferred_element_type=_f32)
                       + jnp.dot(hh, whr[...], preferred_element_type=_f32)
                       + br[...] + cr[...])
    z = jax.nn.sigmoid(jnp.dot(x, wiz[...], preferred_element_type=_f32)
                       + jnp.dot(hh, whz[...], preferred_element_type=_f32)
                       + bz[...] + cz[...])
    n = jnp.tanh(jnp.dot(x, win[...], preferred_element_type=_f32)
                 + bn[...]
                 + r * (jnp.dot(hh, whn[...], preferred_element_type=_f32) + cn[...]))
    hnew = (1.0 - z) * n + z * hh
    out[...] = hnew
    a_out[...] = jnp.dot(hnew, w1a[...], preferred_element_type=_f32)
    b_out[...] = jnp.dot(hnew, w1b[...], preferred_element_type=_f32)


def _readout_body(h, w1, b1, w2, b2, wd, bd, out):
    x = jnp.maximum(jnp.dot(h[...], w1[...], preferred_element_type=_f32) + b1[...], 0.0)
    x = jnp.maximum(jnp.dot(x, w2[...], preferred_element_type=_f32) + b2[...], 0.0)
    d = jnp.sum(x * wd[...], axis=1, keepdims=True) + bd[...]
    sgn = 1.0 - 2.0 * lax.broadcasted_iota(jnp.int32, (N, 2), 1).astype(_f32)
    out[...] = jax.nn.sigmoid(sgn * d)


def _make_tc_calls():
    full = pl.BlockSpec(index_map=lambda i: (0, 0))
    mlp = pl.pallas_call(
        _mlp_body,
        grid=(CAP // EB,),
        in_specs=[
            pl.BlockSpec((EB, HM), lambda i: (i, 0)),
            pl.BlockSpec((EB, HM), lambda i: (i, 0)),
            pl.BlockSpec((EB, 4), lambda i: (i, 0)),
            full, full, full, full, full, full,
        ],
        out_specs=pl.BlockSpec((EB, HM), lambda i: (i, 0)),
        out_shape=jax.ShapeDtypeStruct((CAP, HM), _f32),
    )
    gru = pl.pallas_call(
        _gru_body,
        out_shape=(jax.ShapeDtypeStruct((N, SD), _f32),
                   jax.ShapeDtypeStruct((N, HM), _f32),
                   jax.ShapeDtypeStruct((N, HM), _f32)),
    )
    readout = pl.pallas_call(
        _readout_body,
        out_shape=jax.ShapeDtypeStruct((N, 2), _f32),
    )
    return mlp, gru, readout


def kernel(J, b, W_m1, b_m1, W_m2, b_m2, W_m3, b_m3, W_ih, b_ih, W_hh, b_hh,
           W_r1, b_r1, W_r2, b_r2, W_r3, b_r3):
    flat = J.reshape(-1)
    (eidx,) = jnp.nonzero(flat, size=CAP, fill_value=0)
    cnt = jnp.count_nonzero(flat)
    valid = jnp.arange(CAP) < cnt
    row = (eidx // N).astype(jnp.int32)
    col = (eidx - row * N).astype(jnp.int32)
    vf = valid.astype(_f32)[:, None]
    ef = jnp.stack([b[row], b[col], flat[eidx], J[col, row]], axis=-1) * vf
    spread = (jnp.arange(CAP) % 128).astype(jnp.int32)
    sidx = jnp.where(valid, col, N + spread).astype(jnp.int32)
    row_g = jnp.where(valid, row, spread * 16)
    col_g = jnp.where(valid, col, spread * 16)

    ridx3 = row_g.reshape(NT, NCH, CHUNK)
    cidx3 = col_g.reshape(NT, NCH, CHUNK)
    sidx3 = sidx.reshape(NT, NCH, CHUNK)
    zeros_acc = jnp.zeros((N_ACC, HM), _f32)

    w1a = W_m1[:, 0:SD].T
    w1b = W_m1[:, SD:2 * SD].T
    w1c = W_m1[:, 2 * SD:2 * SD + 4].T
    b1 = b_m1.reshape(1, HM)
    w2 = W_m2.T
    b2 = b_m2.reshape(1, HM)
    w3 = jnp.pad(W_m3.T, ((0, 0), (0, HM - SD)))
    b3 = jnp.pad(b_m3.reshape(1, SD), ((0, 0), (0, HM - SD)))
    wir, wiz, win = (W_ih[0:SD].T, W_ih[SD:2 * SD].T, W_ih[2 * SD:].T)
    whr, whz, whn = (W_hh[0:SD].T, W_hh[SD:2 * SD].T, W_hh[2 * SD:].T)
    br, bz, bn = (b_ih[0:SD].reshape(1, SD), b_ih[SD:2 * SD].reshape(1, SD),
                  b_ih[2 * SD:].reshape(1, SD))
    cr, cz, cn = (b_hh[0:SD].reshape(1, SD), b_hh[SD:2 * SD].reshape(1, SD),
                  b_hh[2 * SD:].reshape(1, SD))
    wr1 = W_r1.T
    br1 = b_r1.reshape(1, -1)
    wr2 = W_r2.T
    br2 = b_r2.reshape(1, -1)
    wd = (W_r3[0] - W_r3[1]).reshape(1, -1)
    bd = (b_r3[0] - b_r3[1]).reshape(1, 1)

    sc_gather, sc_scatter = _make_sc_calls()
    mlp, gru, readout = _make_tc_calls()

    def step(carry, _):
        hidden, A, B = carry
        ar, bc = sc_gather(A, B, ridx3, cidx3)
        msgs = mlp(ar, bc, ef, w1c, b1, w2, b2, w3, b3)
        nm2 = sc_scatter(msgs, sidx3, zeros_acc)
        hidden, A, B = gru(nm2, hidden, wir, wiz, win, whr, whz, whn,
                           br, bz, bn, cr, cz, cn, w1a, w1b)
        return (hidden, A, B), None

    hidden = jnp.zeros((N, SD), _f32)
    A0 = jnp.zeros((N, HM), _f32)
    B0 = jnp.zeros((N, HM), _f32)
    (hidden, _, _), _ = lax.scan(step, (hidden, A0, B0), None, length=N_STEPS)
    return readout(hidden, wr1, br1, wr2, br2, wd, bd)
